# baseline (speedup 1.0000x reference)
import functools

import jax
import jax.numpy as jnp
from jax import lax
from jax.experimental import pallas as pl
from jax.experimental.pallas import tpu as pltpu
from jax.experimental.pallas import tpu_sc as plsc

N_NODES = 10000
D = 128
LANES = 16
NC = 2
NS = 16
EDGE_BLK = 64
IDX_CH = 16


def _mm_body(do_relu_in, two_parts, *refs):
    if two_parts:
        xa, xb, w, b, out = refs
        x = xa[...] + xb[...]
    else:
        xv, w, b, out = refs
        x = xv[...]
    if do_relu_in:
        x = jnp.maximum(x, 0.0)
    y = lax.dot_general(x, w[...], (((1,), (1,)), ((), ())),
                        preferred_element_type=jnp.float32)
    out[...] = y + b[...][None, :]


def _matmul(x_parts, w, b, do_relu_in):
    two = len(x_parts) == 2
    n = x_parts[0].shape[0]
    blk = 2048
    grid = n // blk
    in_specs = [pl.BlockSpec((blk, D), lambda i: (i, 0)) for _ in x_parts]
    in_specs += [pl.BlockSpec((D, D), lambda i: (0, 0)),
                 pl.BlockSpec((D,), lambda i: (0,))]
    return pl.pallas_call(
        functools.partial(_mm_body, do_relu_in, two),
        grid=(grid,),
        in_specs=in_specs,
        out_specs=pl.BlockSpec((blk, D), lambda i: (i, 0)),
        out_shape=jax.ShapeDtypeStruct((n, D), jnp.float32),
    )(*x_parts, w, b)


def _proj_body(ea, eb, wp1, bp1, wp2, bp2, emb_out, z_out):
    e = ea[...] + eb[...]
    emb_out[...] = e
    t = jnp.maximum(
        lax.dot_general(e, wp1[...], (((1,), (1,)), ((), ())),
                        preferred_element_type=jnp.float32) + bp1[...][None, :],
        0.0)
    z_out[...] = lax.dot_general(t, wp2[...], (((1,), (1,)), ((), ())),
                                 preferred_element_type=jnp.float32) \
        + bp2[...][None, :]


def _proj_head(ea, eb, wp1, bp1, wp2, bp2):
    n = ea.shape[0]
    blk = 2048
    grid = n // blk
    vec = pl.BlockSpec((D,), lambda i: (0,))
    mat = pl.BlockSpec((D, D), lambda i: (0, 0))
    rows = pl.BlockSpec((blk, D), lambda i: (i, 0))
    return pl.pallas_call(
        _proj_body,
        grid=(grid,),
        in_specs=[rows, rows, mat, vec, mat, vec],
        out_specs=(rows, rows),
        out_shape=(jax.ShapeDtypeStruct((n, D), jnp.float32),
                   jax.ShapeDtypeStruct((n, D), jnp.float32)),
    )(ea, eb, wp1, bp1, wp2, bp2)


N_PAD = 10240


def _make_seg_kernel(blocks_per_tile):
    nrows = N_PAD
    rows_per_tile = nrows // NS

    mesh = plsc.VectorSubcoreMesh(core_axis_name="c", subcore_axis_name="s")
    CH = IDX_CH
    nbuf = 4

    @functools.partial(
        pl.kernel,
        mesh=mesh,
        out_type=jax.ShapeDtypeStruct((NC, nrows, D), jnp.float32),
        scratch_types=[
            pltpu.VMEM((2 * CH, EDGE_BLK), jnp.int32),
            pltpu.VMEM((2 * CH, EDGE_BLK), jnp.int32),
            pltpu.VMEM((2 * CH, EDGE_BLK), jnp.float32),
            pltpu.VMEM((EDGE_BLK, D), jnp.float32),
            pltpu.VMEM((EDGE_BLK, D), jnp.float32),
            pltpu.VMEM((EDGE_BLK, D), jnp.float32),
            pltpu.VMEM((EDGE_BLK, D), jnp.float32),
            pltpu.VMEM_SHARED((nrows, D), jnp.float32),
            pltpu.SemaphoreType.DMA,
            pltpu.SemaphoreType.DMA,
            pltpu.SemaphoreType.DMA,
            pltpu.SemaphoreType.DMA,
            pltpu.SemaphoreType.DMA,
            pltpu.SemaphoreType.DMA,
            pltpu.SemaphoreType.DMA,
            pltpu.SemaphoreType.DMA,
            pltpu.SemaphoreType.DMA,
        ],
    )
    def seg(h_hbm, src_hbm, dst_hbm, w_hbm, out_hbm,
            src_v, dst_v, w_v, r0, r1, r2, r3, acc_sp,
            g0, g1, g2, g3, s0, s1, s2, s3, isem):
        rows = [r0, r1, r2, r3]
        gsem = [g0, g1, g2, g3]
        ssem = [s0, s1, s2, s3]
        c = lax.axis_index("c")
        s = lax.axis_index("s")
        wid = (1 - c) * NS + s
        nb = blocks_per_tile
        nch = nb // CH
        row0 = wid * nb

        pltpu.sync_copy(src_hbm.at[pl.ds(row0, CH)], src_v.at[pl.ds(0, CH)])
        pltpu.sync_copy(dst_hbm.at[pl.ds(row0, CH)], dst_v.at[pl.ds(0, CH)])
        pltpu.sync_copy(w_hbm.at[pl.ds(row0, CH)], w_v.at[pl.ds(0, CH)])

        for cc in range(EDGE_BLK):
            for ch in range(D // LANES):
                r0[cc, pl.ds(ch * LANES, LANES)] = jnp.zeros(
                    (LANES,), jnp.float32)
        zbase = s * rows_per_tile
        for zz in range(rows_per_tile // EDGE_BLK):
            pltpu.sync_copy(r0.at[pl.ds(0, EDGE_BLK)],
                            acc_sp.at[pl.ds(zbase + zz * EDGE_BLK, EDGE_BLK)])
        plsc.subcore_barrier()

        def gather_start(srow, b):
            pltpu.async_copy(h_hbm.at[src_v.at[srow]], rows[b], gsem[b])

        def gather_wait(b):
            pltpu.make_async_copy(h_hbm.at[src_v.at[0]], rows[b],
                                  gsem[b]).wait()

        def scatter_start(drow, b):
            pltpu.async_copy(rows[b], acc_sp.at[dst_v.at[drow]], ssem[b],
                             add=True)

        def scatter_wait(b):
            pltpu.make_async_copy(rows[b], acc_sp.at[dst_v.at[0]],
                                  ssem[b]).wait()

        def scale(wrow, b):
            def group_body(g, gcarry):
                w16 = w_v[wrow, pl.ds(g * LANES, LANES)]
                for i in range(LANES):
                    wb = jnp.full((LANES,), w16[i])
                    e = g * LANES + i
                    for ch in range(D // LANES):
                        sl = pl.ds(ch * LANES, LANES)
                        rows[b][e, sl] = rows[b][e, sl] * wb
                return gcarry
            lax.fori_loop(0, EDGE_BLK // LANES, group_body, 0)

        gather_start(0, 0)
        gather_start(1, 1)

        def chunk_body(q, carry):
            p = q % 2
            pbase = p * CH
            qbase = (1 - p) * CH
            for k in range(CH):
                b = k % nbuf
                bn = (b + 2) % nbuf

                if k >= 2:
                    scatter_wait(bn)
                else:
                    @pl.when(q > 0)
                    def _():
                        scatter_wait(bn)

                if k == 2:
                    @pl.when(q + 1 < nch)
                    def _():
                        nxt = row0 + (q + 1) * CH
                        pltpu.async_copy(src_hbm.at[pl.ds(nxt, CH)],
                                         src_v.at[pl.ds(qbase, CH)], isem)
                        pltpu.async_copy(dst_hbm.at[pl.ds(nxt, CH)],
                                         dst_v.at[pl.ds(qbase, CH)], isem)
                        pltpu.async_copy(w_hbm.at[pl.ds(nxt, CH)],
                                         w_v.at[pl.ds(qbase, CH)], isem)

                if k == CH - 3:
                    @pl.when(q + 1 < nch)
                    def _():
                        for _i in range(3):
                            pltpu.make_async_copy(
                                src_hbm.at[pl.ds(row0, CH)],
                                src_v.at[pl.ds(qbase, CH)], isem).wait()

                if k < CH - 2:
                    gather_start(pbase + k + 2, bn)
                else:
                    @pl.when(q + 1 < nch)
                    def _():
                        gather_start(qbase + (k + 2 - CH), bn)

                gather_wait(b)
                scale(pbase + k, b)
                scatter_start(pbase + k, b)
            return carry

        lax.fori_loop(0, nch, chunk_body, 0)
        scatter_wait((nb - 2) % nbuf)
        scatter_wait((nb - 1) % nbuf)
        plsc.subcore_barrier()

        obase = s * rows_per_tile
        pltpu.sync_copy(acc_sp.at[pl.ds(obase, rows_per_tile)],
                        out_hbm.at[c].at[pl.ds(obase, rows_per_tile)])

    return seg


def kernel(x, edge_index, edge_w, W1, b1, W2, b2, W3, b3, Wp1, bp1, Wp2, bp2):
    e = edge_index.shape[1]
    blocks_per_tile = -(-e // (NC * NS * EDGE_BLK))
    blocks_per_tile = -(-blocks_per_tile // IDX_CH) * IDX_CH
    e_pad = NC * NS * blocks_per_tile * EDGE_BLK
    pad = e_pad - e

    src = jnp.pad(edge_index[0], (0, pad))
    dst = jnp.pad(edge_index[1], (0, pad))
    w = jnp.pad(edge_w, (0, pad))
    src2 = src.reshape(-1, EDGE_BLK)
    dst2 = dst.reshape(-1, EDGE_BLK)
    w2 = w.reshape(-1, EDGE_BLK)

    seg = _make_seg_kernel(blocks_per_tile)

    n = x.shape[0]
    xp = jnp.pad(x, ((0, N_PAD - n), (0, 0)))

    h1 = _matmul([xp], W1, b1, do_relu_in=False)
    p1 = seg(h1, src2, dst2, w2)
    h2 = _matmul([p1[0], p1[1]], W2, b2, do_relu_in=True)
    p2 = seg(h2, src2, dst2, w2)
    h3 = _matmul([p2[0], p2[1]], W3, b3, do_relu_in=True)
    p3 = seg(h3, src2, dst2, w2)
    emb, z = _proj_head(p3[0], p3[1], Wp1, bp1, Wp2, bp2)
    return (z[:n], emb[:n])


# speedup vs baseline: 2.9033x; 2.9033x over previous
import functools

import jax
import jax.numpy as jnp
from jax import lax
from jax.experimental import pallas as pl
from jax.experimental.pallas import tpu as pltpu
from jax.experimental.pallas import tpu_sc as plsc

N_NODES = 10000
D = 128
LANES = 16
NC = 2
NS = 16
EDGE_BLK = 64
IDX_CH = 16


def _mm_body(do_relu_in, two_parts, *refs):
    if two_parts:
        xa, xb, w, b, out = refs
        x = xa[...] + xb[...]
    else:
        xv, w, b, out = refs
        x = xv[...]
    if do_relu_in:
        x = jnp.maximum(x, 0.0)
    y = lax.dot_general(x, w[...], (((1,), (1,)), ((), ())),
                        preferred_element_type=jnp.float32)
    out[...] = y + b[...][None, :]


def _matmul(x_parts, w, b, do_relu_in):
    two = len(x_parts) == 2
    n = x_parts[0].shape[0]
    blk = 2048
    grid = n // blk
    in_specs = [pl.BlockSpec((blk, D), lambda i: (i, 0)) for _ in x_parts]
    in_specs += [pl.BlockSpec((D, D), lambda i: (0, 0)),
                 pl.BlockSpec((D,), lambda i: (0,))]
    return pl.pallas_call(
        functools.partial(_mm_body, do_relu_in, two),
        grid=(grid,),
        in_specs=in_specs,
        out_specs=pl.BlockSpec((blk, D), lambda i: (i, 0)),
        out_shape=jax.ShapeDtypeStruct((n, D), jnp.float32),
    )(*x_parts, w, b)


def _proj_body(ea, eb, wp1, bp1, wp2, bp2, emb_out, z_out):
    e = ea[...] + eb[...]
    emb_out[...] = e
    t = jnp.maximum(
        lax.dot_general(e, wp1[...], (((1,), (1,)), ((), ())),
                        preferred_element_type=jnp.float32) + bp1[...][None, :],
        0.0)
    z_out[...] = lax.dot_general(t, wp2[...], (((1,), (1,)), ((), ())),
                                 preferred_element_type=jnp.float32) \
        + bp2[...][None, :]


def _proj_head(ea, eb, wp1, bp1, wp2, bp2):
    n = ea.shape[0]
    blk = 2048
    grid = n // blk
    vec = pl.BlockSpec((D,), lambda i: (0,))
    mat = pl.BlockSpec((D, D), lambda i: (0, 0))
    rows = pl.BlockSpec((blk, D), lambda i: (i, 0))
    return pl.pallas_call(
        _proj_body,
        grid=(grid,),
        in_specs=[rows, rows, mat, vec, mat, vec],
        out_specs=(rows, rows),
        out_shape=(jax.ShapeDtypeStruct((n, D), jnp.float32),
                   jax.ShapeDtypeStruct((n, D), jnp.float32)),
    )(ea, eb, wp1, bp1, wp2, bp2)


N_PAD = 10240


def _make_seg_kernel(blocks_per_tile):
    nrows = N_PAD
    rows_per_tile = nrows // NS

    mesh = plsc.VectorSubcoreMesh(core_axis_name="c", subcore_axis_name="s")
    CH = IDX_CH
    nbuf = 4

    @functools.partial(
        pl.kernel,
        mesh=mesh,
        out_type=jax.ShapeDtypeStruct((NC, nrows, D), jnp.float32),
        scratch_types=[
            pltpu.VMEM((2 * CH, EDGE_BLK), jnp.int32),
            pltpu.VMEM((2 * CH, EDGE_BLK), jnp.int32),
            pltpu.VMEM((2 * CH, EDGE_BLK), jnp.float32),
            pltpu.VMEM((EDGE_BLK, D), jnp.float32),
            pltpu.VMEM((EDGE_BLK, D), jnp.float32),
            pltpu.VMEM((EDGE_BLK, D), jnp.float32),
            pltpu.VMEM((EDGE_BLK, D), jnp.float32),
            pltpu.VMEM_SHARED((nrows, D), jnp.float32),
            pltpu.SemaphoreType.DMA,
            pltpu.SemaphoreType.DMA,
            pltpu.SemaphoreType.DMA,
            pltpu.SemaphoreType.DMA,
            pltpu.SemaphoreType.DMA,
            pltpu.SemaphoreType.DMA,
            pltpu.SemaphoreType.DMA,
            pltpu.SemaphoreType.DMA,
            pltpu.SemaphoreType.DMA,
        ],
    )
    def seg(h_hbm, src_hbm, dst_hbm, w_hbm, out_hbm,
            src_v, dst_v, w_v, r0, r1, r2, r3, acc_sp,
            g0, g1, g2, g3, s0, s1, s2, s3, isem):
        rows = [r0, r1, r2, r3]
        gsem = [g0, g1, g2, g3]
        ssem = [s0, s1, s2, s3]
        c = lax.axis_index("c")
        s = lax.axis_index("s")
        wid = c * NS + s
        nb = blocks_per_tile
        nch = nb // CH
        row0 = wid * nb

        pltpu.sync_copy(src_hbm.at[pl.ds(row0, CH)], src_v.at[pl.ds(0, CH)])
        pltpu.sync_copy(dst_hbm.at[pl.ds(row0, CH)], dst_v.at[pl.ds(0, CH)])
        pltpu.sync_copy(w_hbm.at[pl.ds(row0, CH)], w_v.at[pl.ds(0, CH)])

        for cc in range(EDGE_BLK):
            for ch in range(D // LANES):
                r0[cc, pl.ds(ch * LANES, LANES)] = jnp.zeros(
                    (LANES,), jnp.float32)
        zbase = s * rows_per_tile
        for zz in range(rows_per_tile // EDGE_BLK):
            pltpu.sync_copy(r0.at[pl.ds(0, EDGE_BLK)],
                            acc_sp.at[pl.ds(zbase + zz * EDGE_BLK, EDGE_BLK)])
        plsc.subcore_barrier()

        def gather_start(srow, b):
            pltpu.async_copy(h_hbm.at[src_v.at[srow]], rows[b], gsem[b])

        def gather_wait(b):
            pltpu.make_async_copy(h_hbm.at[src_v.at[0]], rows[b],
                                  gsem[b]).wait()

        def scatter_start(drow, b):
            pltpu.async_copy(rows[b], acc_sp.at[dst_v.at[drow]], ssem[b],
                             add=True)

        def scatter_wait(b):
            pltpu.make_async_copy(rows[b], acc_sp.at[dst_v.at[0]],
                                  ssem[b]).wait()

        def scale(wrow, b):
            def group_body(g, gcarry):
                w16 = w_v[wrow, pl.ds(g * LANES, LANES)]
                for i in range(LANES):
                    wb = jnp.full((LANES,), w16[i])
                    e = g * LANES + i
                    for ch in range(D // LANES):
                        sl = pl.ds(ch * LANES, LANES)
                        rows[b][e, sl] = rows[b][e, sl] * wb
                return gcarry
            lax.fori_loop(0, EDGE_BLK // LANES, group_body, 0)

        gather_start(0, 0)
        gather_start(1, 1)

        def chunk_body(q, carry):
            p = q % 2
            pbase = p * CH
            qbase = (1 - p) * CH
            for k in range(CH):
                b = k % nbuf
                bn = (b + 2) % nbuf

                if k >= 2:
                    scatter_wait(bn)
                else:
                    @pl.when(q > 0)
                    def _():
                        scatter_wait(bn)

                if k == 2:
                    @pl.when(q + 1 < nch)
                    def _():
                        nxt = row0 + (q + 1) * CH
                        pltpu.async_copy(src_hbm.at[pl.ds(nxt, CH)],
                                         src_v.at[pl.ds(qbase, CH)], isem)
                        pltpu.async_copy(dst_hbm.at[pl.ds(nxt, CH)],
                                         dst_v.at[pl.ds(qbase, CH)], isem)
                        pltpu.async_copy(w_hbm.at[pl.ds(nxt, CH)],
                                         w_v.at[pl.ds(qbase, CH)], isem)

                if k == CH - 3:
                    @pl.when(q + 1 < nch)
                    def _():
                        for _i in range(3):
                            pltpu.make_async_copy(
                                src_hbm.at[pl.ds(row0, CH)],
                                src_v.at[pl.ds(qbase, CH)], isem).wait()

                if k < CH - 2:
                    gather_start(pbase + k + 2, bn)
                else:
                    @pl.when(q + 1 < nch)
                    def _():
                        gather_start(qbase + (k + 2 - CH), bn)

                gather_wait(b)
                scale(pbase + k, b)
                scatter_start(pbase + k, b)
            return carry

        lax.fori_loop(0, nch, chunk_body, 0)
        scatter_wait((nb - 2) % nbuf)
        scatter_wait((nb - 1) % nbuf)
        plsc.subcore_barrier()

        obase = s * rows_per_tile
        pltpu.sync_copy(acc_sp.at[pl.ds(obase, rows_per_tile)],
                        out_hbm.at[c].at[pl.ds(obase, rows_per_tile)])

    return seg


def kernel(x, edge_index, edge_w, W1, b1, W2, b2, W3, b3, Wp1, bp1, Wp2, bp2):
    e = edge_index.shape[1]
    blocks_per_tile = -(-e // (NC * NS * EDGE_BLK))
    blocks_per_tile = -(-blocks_per_tile // IDX_CH) * IDX_CH
    e_pad = NC * NS * blocks_per_tile * EDGE_BLK
    pad = e_pad - e

    spread = (jnp.arange(pad, dtype=jnp.int32) * 8) % N_PAD
    src = jnp.concatenate([edge_index[0], spread])
    dst = jnp.concatenate([edge_index[1], spread])
    w = jnp.pad(edge_w, (0, pad))
    src2 = src.reshape(-1, EDGE_BLK)
    dst2 = dst.reshape(-1, EDGE_BLK)
    w2 = w.reshape(-1, EDGE_BLK)

    seg = _make_seg_kernel(blocks_per_tile)

    n = x.shape[0]
    xp = jnp.pad(x, ((0, N_PAD - n), (0, 0)))

    h1 = _matmul([xp], W1, b1, do_relu_in=False)
    p1 = seg(h1, src2, dst2, w2)
    h2 = _matmul([p1[0], p1[1]], W2, b2, do_relu_in=True)
    p2 = seg(h2, src2, dst2, w2)
    h3 = _matmul([p2[0], p2[1]], W3, b3, do_relu_in=True)
    p3 = seg(h3, src2, dst2, w2)
    emb, z = _proj_head(p3[0], p3[1], Wp1, bp1, Wp2, bp2)
    return (z[:n], emb[:n])
